# baseline (speedup 1.0000x reference)
import functools

import jax
import jax.numpy as jnp
from jax import lax
from jax.experimental import pallas as pl
from jax.experimental.pallas import tpu as pltpu
from jax.experimental.pallas import tpu_sc as plsc

B = 16384
NU = 1000000
D_GMF = 32
D_MLP = 64
NG = 26
GP = 16
H1 = 128
H2 = 64

_NC = 2
_NS = 16
_NW = _NC * _NS
_BPW = B // _NW
_NBLK = NU // 512
_BLK_PER_W = _NBLK // _NW
_TSTART = NU - 64
_CHUNK = 2048
_SROWS = 32


def _scan_compact(idx_hbm, chunk_v, todo, jb0, jb1, sem):
    nch = B // _CHUNK

    def outer(q, n):
        pltpu.async_copy(idx_hbm.at[pl.ds(q * _CHUNK, _CHUNK)],
                         chunk_v, sem).wait()

        def body(kk, n):
            r = chunk_v[pl.ds(kk * 16, 16)]
            bvec = q * _CHUNK + kk * 16 + lax.iota(jnp.int32, 16)
            blk = lax.shift_right_logical(r, 9)
            mine = (blk >= jb0) & (blk < jb1)
            packed = (lax.shift_left(blk - jb0, 24)
                      | lax.shift_left(r & 511, 14) | bvec)
            pos = n + plsc.cumsum(mine.astype(jnp.int32)) - 1
            plsc.store_scatter(todo, [pos], packed, mask=mine)
            return n + plsc.all_reduce_population_count(mine)

        return lax.fori_loop(0, _CHUNK // 16, body, n)

    n = lax.fori_loop(0, nch, outer, jnp.zeros((16,), jnp.int32))
    return jnp.max(n)


def _sweep(gT, mT, out_hbm, todo, pans_g, pans_m, stage, sb, sem, psem,
           jb0, jb1, n_s):
    nchunk = (n_s + 15) // 16
    cvec16 = lax.iota(jnp.int32, 16)

    def reset_sb():
        for q in range(_SROWS // 16):
            sb[pl.ds(q * 16, 16)] = jnp.full((16,), -1, jnp.int32)

    reset_sb()

    def flush():
        cp = pltpu.make_async_copy(
            stage, out_hbm.at[plsc.Indices(sb, ignored_value=-1)], sem)
        cp.start()
        cp.wait()

    def fetch(jb, slot):
        off = pl.multiple_of(jb * 512, 128)
        pltpu.make_async_copy(
            gT.at[:, pl.ds(off, 512)], pans_g.at[slot], psem).start()
        pltpu.make_async_copy(
            mT.at[:, pl.ds(off, 512)], pans_m.at[slot], psem).start()

    def wait_fetch(slot):
        pltpu.make_async_copy(
            gT.at[:, pl.ds(0, 512)], pans_g.at[slot], psem).wait()
        pltpu.make_async_copy(
            mT.at[:, pl.ds(0, 512)], pans_m.at[slot], psem).wait()

    def process_block(jb, slot, m):
        jb_rel = jb - jb0

        def chunk_body(kk, m):
            pk = todo[pl.ds(kk * 16, 16)]
            valid = cvec16 < (n_s - kk * 16)
            pm = valid & (lax.shift_right_logical(pk, 24) == jb_rel)
            hv = plsc.all_reduce_population_count(pm)

            @pl.when(hv[0] > 0)
            def _():
                col = lax.shift_right_logical(pk, 14) & 511
                bb = pk & 16383
                rows = m + plsc.cumsum(pm.astype(jnp.int32)) - 1
                pg = pans_g.at[slot]
                pmm = pans_m.at[slot]

                def gmf_feat(c, _):
                    cv = jnp.full((16,), c, jnp.int32)
                    vals = plsc.load_gather(pg, [cv, col], mask=pm)
                    plsc.store_scatter(stage, [rows, cv], vals, mask=pm)
                    return 0

                lax.fori_loop(0, D_GMF, gmf_feat, 0)

                def mlp_feat(c, _):
                    cv = jnp.full((16,), c, jnp.int32)
                    vals = plsc.load_gather(pmm, [cv, col], mask=pm)
                    plsc.store_scatter(
                        stage, [rows, cv + D_GMF], vals, mask=pm)
                    return 0

                lax.fori_loop(0, D_MLP, mlp_feat, 0)
                plsc.store_scatter(sb, [rows], bb, mask=pm)

            m = m + hv
            m_s = m[0]

            @pl.when(m_s >= _SROWS - 16)
            def _():
                flush()
                reset_sb()

            return jnp.where(m_s >= _SROWS - 16,
                             jnp.zeros((16,), jnp.int32), m)

        return lax.fori_loop(0, nchunk, chunk_body, m)

    fetch(jb0, 0)

    def pair_body(p, m):
        jb = jb0 + p * 2
        wait_fetch(0)
        fetch(jnp.minimum(jb + 1, _NBLK - 1), 1)
        m = process_block(jb, 0, m)
        wait_fetch(1)
        fetch(jnp.minimum(jb + 2, _NBLK - 1), 0)
        m = process_block(jb + 1, 1, m)
        return m

    lax.fori_loop(0, (_BLK_PER_W + 2) // 2, pair_body,
                  jnp.zeros((16,), jnp.int32))
    wait_fetch(0)
    flush()
    reset_sb()


def _tail(idx_hbm, tail_hbm, out_hbm, base, chunk_v, stage, tidx, tsb, sem):
    pltpu.async_copy(idx_hbm.at[pl.ds(base, _BPW)], chunk_v.at[pl.ds(0, _BPW)],
                     sem).wait()

    def tail_chunk(q, _):
        def setup(kk, _):
            i = q * _SROWS + kk * 16
            r = chunk_v[pl.ds(i, 16)]
            bvec = base + i + lax.iota(jnp.int32, 16)
            pos = kk * 16 + lax.iota(jnp.int32, 16)
            plsc.store_scatter(tidx, [pos], jnp.maximum(r - _TSTART, 0))
            plsc.store_scatter(tsb, [pos],
                               jnp.where(r >= _TSTART, bvec, -1))
            return 0

        lax.fori_loop(0, _SROWS // 16, setup, 0)
        pltpu.async_copy(tail_hbm.at[tidx], stage, sem).wait()
        cp = pltpu.make_async_copy(
            stage, out_hbm.at[plsc.Indices(tsb, ignored_value=-1)], sem)
        cp.start()
        cp.wait()
        return 0

    lax.fori_loop(0, _BPW // _SROWS, tail_chunk, 0)


def _sc_sweep_gather(user_ids, item_ids, ugT, umT, igT, imT, tail_u, tail_i):
    mesh = plsc.VectorSubcoreMesh(core_axis_name="c", subcore_axis_name="s")

    @functools.partial(
        pl.kernel,
        mesh=mesh,
        compiler_params=pltpu.CompilerParams(
            use_tc_tiling_on_sc=True, needs_layout_passes=False),
        out_type=[
            jax.ShapeDtypeStruct((B, 128), jnp.float32),
            jax.ShapeDtypeStruct((B, 128), jnp.float32),
        ],
        scratch_types=[
            pltpu.VMEM((_CHUNK,), jnp.int32),
            pltpu.VMEM((B,), jnp.int32),
            pltpu.VMEM((2, D_GMF, 512), jnp.float32),
            pltpu.VMEM((2, D_MLP, 512), jnp.float32),
            pltpu.VMEM((_SROWS, 128), jnp.float32),
            pltpu.VMEM((_SROWS,), jnp.int32),
            pltpu.VMEM((_SROWS,), jnp.int32),
            pltpu.SemaphoreType.DMA,
            pltpu.SemaphoreType.DMA,
        ],
    )
    def k(uid_hbm, iid_hbm, ugT_h, umT_h, igT_h, imT_h, tu_h, ti_h,
          out_u, out_i,
          chunk_v, todo, pans_g, pans_m, stage, sb, tidx, sem, psem):
        wid = lax.axis_index("s") * _NC + lax.axis_index("c")
        jb0 = wid * _BLK_PER_W
        jb1 = jnp.where(wid == _NW - 1, _NBLK, jb0 + _BLK_PER_W)
        base = wid * _BPW

        n_s = _scan_compact(uid_hbm, chunk_v, todo, jb0, jb1, sem)
        _sweep(ugT_h, umT_h, out_u, todo, pans_g, pans_m, stage, sb,
               sem, psem, jb0, jb1, n_s)
        _tail(uid_hbm, tu_h, out_u, base, chunk_v, stage, tidx, sb, sem)

        n_s = _scan_compact(iid_hbm, chunk_v, todo, jb0, jb1, sem)
        _sweep(igT_h, imT_h, out_i, todo, pans_g, pans_m,
               stage, sb, sem, psem, jb0, jb1, n_s)
        _tail(iid_hbm, ti_h, out_i, base, chunk_v, stage, tidx, sb, sem)

    return k(user_ids, item_ids, ugT, umT, igT, imT, tail_u, tail_i)


_DBLK = 2048


def _dense_body(uo_ref, io_ref, gn_ref, gW_ref, gb_ref,
                W1_ref, b1_ref, W2_ref, b2_ref, Wf_ref, bf_ref, out_ref):
    mu = uo_ref[:, D_GMF:D_GMF + D_MLP]
    mi = io_ref[:, D_GMF:D_GMF + D_MLP]
    ge = jnp.dot(gn_ref[:], gW_ref[:],
                 preferred_element_type=jnp.float32) + gb_ref[:]
    h = (jnp.dot(mu, W1_ref[0:D_MLP, :],
                 preferred_element_type=jnp.float32)
         + jnp.dot(mi, W1_ref[D_MLP:2 * D_MLP, :],
                   preferred_element_type=jnp.float32)
         + jnp.dot(ge, W1_ref[2 * D_MLP:2 * D_MLP + GP, :],
                   preferred_element_type=jnp.float32)
         + b1_ref[:])
    h = jnp.maximum(h, 0.0)
    h2 = jnp.maximum(
        jnp.dot(h, W2_ref[:], preferred_element_type=jnp.float32) + b2_ref[:],
        0.0)
    gmf = uo_ref[:, 0:D_GMF] * io_ref[:, 0:D_GMF]
    out_ref[:] = (jnp.dot(gmf, Wf_ref[0:D_GMF, :],
                          preferred_element_type=jnp.float32)
                  + jnp.dot(h2, Wf_ref[D_GMF:D_GMF + H2, :],
                            preferred_element_type=jnp.float32)
                  + bf_ref[:])


def _dense(out_u, out_i, genres, genre_W, genre_b, W1, b1, W2, b2, Wf, bf):
    grid = (B // _DBLK,)
    row = lambda i: (i, 0)
    rep = lambda i: (0, 0)
    out = pl.pallas_call(
        _dense_body,
        grid=grid,
        in_specs=[
            pl.BlockSpec((_DBLK, 128), row),
            pl.BlockSpec((_DBLK, 128), row),
            pl.BlockSpec((_DBLK, NG), row),
            pl.BlockSpec((NG, GP), rep),
            pl.BlockSpec((1, GP), rep),
            pl.BlockSpec((2 * D_MLP + GP, H1), rep),
            pl.BlockSpec((1, H1), rep),
            pl.BlockSpec((H1, H2), rep),
            pl.BlockSpec((1, H2), rep),
            pl.BlockSpec((D_GMF + H2, 1), rep),
            pl.BlockSpec((1, 1), rep),
        ],
        out_specs=pl.BlockSpec((_DBLK, 1), row),
        out_shape=jax.ShapeDtypeStruct((B, 1), jnp.float32),
    )(out_u, out_i, genres,
      genre_W, genre_b.reshape(1, GP),
      W1, b1.reshape(1, H1), W2, b2.reshape(1, H2),
      Wf, bf.reshape(1, 1))
    return out[:, 0]


def kernel(user_ids, item_ids, genres, user_gmf, item_gmf, user_mlp,
           item_mlp, genre_W, genre_b, W1, b1, W2, b2, Wf, bf):
    pad = jnp.zeros((64, 128 - D_GMF - D_MLP), jnp.float32)
    tail_u = jnp.concatenate(
        [user_gmf[_TSTART:], user_mlp[_TSTART:], pad], axis=1)
    tail_i = jnp.concatenate(
        [item_gmf[_TSTART:], item_mlp[_TSTART:], pad], axis=1)
    out_u, out_i = _sc_sweep_gather(
        user_ids, item_ids,
        user_gmf.T, user_mlp.T, item_gmf.T, item_mlp.T, tail_u, tail_i)
    return _dense(out_u, out_i, genres, genre_W, genre_b,
                  W1, b1, W2, b2, Wf, bf)


# speedup vs baseline: 1.0884x; 1.0884x over previous
import functools

import jax
import jax.numpy as jnp
from jax import lax
from jax.experimental import pallas as pl
from jax.experimental.pallas import tpu as pltpu
from jax.experimental.pallas import tpu_sc as plsc

B = 16384
NU = 1000000
D_GMF = 32
D_MLP = 64
NG = 26
GP = 16
H1 = 128
H2 = 64

_NC = 2
_NS = 16
_NW = _NC * _NS
_BPW = B // _NW
_NBLK = NU // 512
_BLK_PER_W = _NBLK // _NW
_TSTART = NU - 64
_CHUNK = 2048
_SROWS = 32


def _scan_compact(idx_hbm, chunk_v, todo, jb0, jb1, sem):
    nch = B // _CHUNK

    def outer(q, ns):
        pltpu.async_copy(idx_hbm.at[pl.ds(q * _CHUNK, _CHUNK)],
                         chunk_v, sem).wait()

        def body(kk, ns):
            ne, no = ns
            r = chunk_v[pl.ds(kk * 16, 16)]
            bvec = q * _CHUNK + kk * 16 + lax.iota(jnp.int32, 16)
            blk = lax.shift_right_logical(r, 9)
            mine = (blk >= jb0) & (blk < jb1)
            odd = (blk & 1) == 1
            me = mine & jnp.logical_not(odd)
            mo = mine & odd
            packed = (lax.shift_left(blk - jb0, 24)
                      | lax.shift_left(r & 511, 14) | bvec)
            pe = ne + plsc.cumsum(me.astype(jnp.int32)) - 1
            po = (B - 1) - (no + plsc.cumsum(mo.astype(jnp.int32)) - 1)
            plsc.store_scatter(todo, [pe], packed, mask=me)
            plsc.store_scatter(todo, [po], packed, mask=mo)
            return (ne + plsc.all_reduce_population_count(me),
                    no + plsc.all_reduce_population_count(mo))

        return lax.fori_loop(0, _CHUNK // 16, body, ns)

    z16 = jnp.zeros((16,), jnp.int32)
    ne, no = lax.fori_loop(0, nch, outer, (z16, z16))
    return jnp.max(ne), jnp.max(no)


def _sweep(gT, mT, out_hbm, todo, pans_g, pans_m, stage, sb, sem, psem,
           jb0, jb1, ne_s, no_s):
    nch_e = (ne_s + 15) // 16
    nch_o = (no_s + 15) // 16
    cvec16 = lax.iota(jnp.int32, 16)

    def reset_sb():
        for q in range(_SROWS // 16):
            sb[pl.ds(q * 16, 16)] = jnp.full((16,), -1, jnp.int32)

    reset_sb()

    def flush():
        cp = pltpu.make_async_copy(
            stage, out_hbm.at[plsc.Indices(sb, ignored_value=-1)], sem)
        cp.start()
        cp.wait()

    def fetch(jb, slot):
        off = pl.multiple_of(jb * 512, 128)
        pltpu.make_async_copy(
            gT.at[:, pl.ds(off, 512)], pans_g.at[slot], psem).start()
        pltpu.make_async_copy(
            mT.at[:, pl.ds(off, 512)], pans_m.at[slot], psem).start()

    def wait_fetch(slot):
        pltpu.make_async_copy(
            gT.at[:, pl.ds(0, 512)], pans_g.at[slot], psem).wait()
        pltpu.make_async_copy(
            mT.at[:, pl.ds(0, 512)], pans_m.at[slot], psem).wait()

    def process_block(jb, slot, m):
        jb_rel = jb - jb0
        is_odd = (jb & 1) == 1
        nchunk = jnp.where(is_odd, nch_o, nch_e)

        def chunk_body(kk, m):
            off = jnp.where(is_odd, B - (kk + 1) * 16, kk * 16)
            pk = todo[pl.ds(off, 16)]
            valid = jnp.where(
                is_odd,
                cvec16 >= (16 - (no_s - kk * 16)),
                cvec16 < (ne_s - kk * 16))
            pm = valid & (lax.shift_right_logical(pk, 24) == jb_rel)
            hv = plsc.all_reduce_population_count(pm)

            @pl.when(hv[0] > 0)
            def _():
                col = lax.shift_right_logical(pk, 14) & 511
                bb = pk & 16383
                rows = m + plsc.cumsum(pm.astype(jnp.int32)) - 1
                pg = pans_g.at[slot]
                pmm = pans_m.at[slot]
                for c in range(D_GMF):
                    cv = jnp.full((16,), c, jnp.int32)
                    vals = plsc.load_gather(pg, [cv, col], mask=pm)
                    plsc.store_scatter(stage, [rows, cv], vals, mask=pm)
                for c in range(D_MLP):
                    cv = jnp.full((16,), c, jnp.int32)
                    vals = plsc.load_gather(pmm, [cv, col], mask=pm)
                    plsc.store_scatter(
                        stage, [rows, cv + D_GMF], vals, mask=pm)
                plsc.store_scatter(sb, [rows], bb, mask=pm)

            m = m + hv
            m_s = m[0]

            @pl.when(m_s >= _SROWS - 16)
            def _():
                flush()
                reset_sb()

            return jnp.where(m_s >= _SROWS - 16,
                             jnp.zeros((16,), jnp.int32), m)

        return lax.fori_loop(0, nchunk, chunk_body, m)

    fetch(jb0, 0)

    def pair_body(p, m):
        jb = jb0 + p * 2
        wait_fetch(0)
        fetch(jnp.minimum(jb + 1, _NBLK - 1), 1)
        m = process_block(jb, 0, m)
        wait_fetch(1)
        fetch(jnp.minimum(jb + 2, _NBLK - 1), 0)
        m = process_block(jb + 1, 1, m)
        return m

    lax.fori_loop(0, (_BLK_PER_W + 2) // 2, pair_body,
                  jnp.zeros((16,), jnp.int32))
    wait_fetch(0)
    flush()
    reset_sb()


def _tail(idx_hbm, tail_hbm, out_hbm, base, chunk_v, stage, tidx, tsb, sem):
    pltpu.async_copy(idx_hbm.at[pl.ds(base, _BPW)], chunk_v.at[pl.ds(0, _BPW)],
                     sem).wait()

    def tail_chunk(q, _):
        def setup(kk, _):
            i = q * _SROWS + kk * 16
            r = chunk_v[pl.ds(i, 16)]
            bvec = base + i + lax.iota(jnp.int32, 16)
            pos = kk * 16 + lax.iota(jnp.int32, 16)
            plsc.store_scatter(tidx, [pos], jnp.maximum(r - _TSTART, 0))
            plsc.store_scatter(tsb, [pos],
                               jnp.where(r >= _TSTART, bvec, -1))
            return 0

        lax.fori_loop(0, _SROWS // 16, setup, 0)
        pltpu.async_copy(tail_hbm.at[tidx], stage, sem).wait()
        cp = pltpu.make_async_copy(
            stage, out_hbm.at[plsc.Indices(tsb, ignored_value=-1)], sem)
        cp.start()
        cp.wait()
        return 0

    lax.fori_loop(0, _BPW // _SROWS, tail_chunk, 0)


def _sc_sweep_gather(user_ids, item_ids, ugT, umT, igT, imT, tail_u, tail_i):
    mesh = plsc.VectorSubcoreMesh(core_axis_name="c", subcore_axis_name="s")

    @functools.partial(
        pl.kernel,
        mesh=mesh,
        compiler_params=pltpu.CompilerParams(
            use_tc_tiling_on_sc=True, needs_layout_passes=False),
        out_type=[
            jax.ShapeDtypeStruct((B, 128), jnp.float32),
            jax.ShapeDtypeStruct((B, 128), jnp.float32),
        ],
        scratch_types=[
            pltpu.VMEM((_CHUNK,), jnp.int32),
            pltpu.VMEM((B,), jnp.int32),
            pltpu.VMEM((2, D_GMF, 512), jnp.float32),
            pltpu.VMEM((2, D_MLP, 512), jnp.float32),
            pltpu.VMEM((_SROWS, 128), jnp.float32),
            pltpu.VMEM((_SROWS,), jnp.int32),
            pltpu.VMEM((_SROWS,), jnp.int32),
            pltpu.SemaphoreType.DMA,
            pltpu.SemaphoreType.DMA,
        ],
    )
    def k(uid_hbm, iid_hbm, ugT_h, umT_h, igT_h, imT_h, tu_h, ti_h,
          out_u, out_i,
          chunk_v, todo, pans_g, pans_m, stage, sb, tidx, sem, psem):
        wid = lax.axis_index("s") * _NC + lax.axis_index("c")
        jb0 = wid * _BLK_PER_W
        jb1 = jnp.where(wid == _NW - 1, _NBLK, jb0 + _BLK_PER_W)
        base = wid * _BPW

        ne_s, no_s = _scan_compact(uid_hbm, chunk_v, todo, jb0, jb1, sem)
        _sweep(ugT_h, umT_h, out_u, todo, pans_g, pans_m, stage, sb,
               sem, psem, jb0, jb1, ne_s, no_s)
        _tail(uid_hbm, tu_h, out_u, base, chunk_v, stage, tidx, sb, sem)

        ne_s, no_s = _scan_compact(iid_hbm, chunk_v, todo, jb0, jb1, sem)
        _sweep(igT_h, imT_h, out_i, todo, pans_g, pans_m,
               stage, sb, sem, psem, jb0, jb1, ne_s, no_s)
        _tail(iid_hbm, ti_h, out_i, base, chunk_v, stage, tidx, sb, sem)

    return k(user_ids, item_ids, ugT, umT, igT, imT, tail_u, tail_i)


_DBLK = 2048


def _dense_body(uo_ref, io_ref, gn_ref, gW_ref, gb_ref,
                W1_ref, b1_ref, W2_ref, b2_ref, Wf_ref, bf_ref, out_ref):
    mu = uo_ref[:, D_GMF:D_GMF + D_MLP]
    mi = io_ref[:, D_GMF:D_GMF + D_MLP]
    ge = jnp.dot(gn_ref[:], gW_ref[:],
                 preferred_element_type=jnp.float32) + gb_ref[:]
    h = (jnp.dot(mu, W1_ref[0:D_MLP, :],
                 preferred_element_type=jnp.float32)
         + jnp.dot(mi, W1_ref[D_MLP:2 * D_MLP, :],
                   preferred_element_type=jnp.float32)
         + jnp.dot(ge, W1_ref[2 * D_MLP:2 * D_MLP + GP, :],
                   preferred_element_type=jnp.float32)
         + b1_ref[:])
    h = jnp.maximum(h, 0.0)
    h2 = jnp.maximum(
        jnp.dot(h, W2_ref[:], preferred_element_type=jnp.float32) + b2_ref[:],
        0.0)
    gmf = uo_ref[:, 0:D_GMF] * io_ref[:, 0:D_GMF]
    out_ref[:] = (jnp.dot(gmf, Wf_ref[0:D_GMF, :],
                          preferred_element_type=jnp.float32)
                  + jnp.dot(h2, Wf_ref[D_GMF:D_GMF + H2, :],
                            preferred_element_type=jnp.float32)
                  + bf_ref[:])


def _dense(out_u, out_i, genres, genre_W, genre_b, W1, b1, W2, b2, Wf, bf):
    grid = (B // _DBLK,)
    row = lambda i: (i, 0)
    rep = lambda i: (0, 0)
    out = pl.pallas_call(
        _dense_body,
        grid=grid,
        in_specs=[
            pl.BlockSpec((_DBLK, 128), row),
            pl.BlockSpec((_DBLK, 128), row),
            pl.BlockSpec((_DBLK, NG), row),
            pl.BlockSpec((NG, GP), rep),
            pl.BlockSpec((1, GP), rep),
            pl.BlockSpec((2 * D_MLP + GP, H1), rep),
            pl.BlockSpec((1, H1), rep),
            pl.BlockSpec((H1, H2), rep),
            pl.BlockSpec((1, H2), rep),
            pl.BlockSpec((D_GMF + H2, 1), rep),
            pl.BlockSpec((1, 1), rep),
        ],
        out_specs=pl.BlockSpec((_DBLK, 1), row),
        out_shape=jax.ShapeDtypeStruct((B, 1), jnp.float32),
    )(out_u, out_i, genres,
      genre_W, genre_b.reshape(1, GP),
      W1, b1.reshape(1, H1), W2, b2.reshape(1, H2),
      Wf, bf.reshape(1, 1))
    return out[:, 0]


def kernel(user_ids, item_ids, genres, user_gmf, item_gmf, user_mlp,
           item_mlp, genre_W, genre_b, W1, b1, W2, b2, Wf, bf):
    pad = jnp.zeros((64, 128 - D_GMF - D_MLP), jnp.float32)
    tail_u = jnp.concatenate(
        [user_gmf[_TSTART:], user_mlp[_TSTART:], pad], axis=1)
    tail_i = jnp.concatenate(
        [item_gmf[_TSTART:], item_mlp[_TSTART:], pad], axis=1)
    out_u, out_i = _sc_sweep_gather(
        user_ids, item_ids,
        user_gmf.T, user_mlp.T, item_gmf.T, item_mlp.T, tail_u, tail_i)
    return _dense(out_u, out_i, genres, genre_W, genre_b,
                  W1, b1, W2, b2, Wf, bf)
